# baseline (speedup 1.0000x reference)
import jax
import jax.numpy as jnp
from jax.experimental import pallas as pl
from jax.experimental.pallas import tpu as pltpu


_ROWS = 16384
_DIM = 128
_NCHUNK = 8
_CHUNK = _ROWS // _NCHUNK


def _copy_kernel(q_ref, out_ref, mask_ref, sems):
    copies = []
    for c in range(_NCHUNK):
        rows = pl.ds(c * _CHUNK, _CHUNK)
        cp = pltpu.make_async_copy(q_ref.at[rows], out_ref.at[rows], sems.at[c])
        cp.start()
        copies.append(cp)
    mask_ref[...] = jnp.ones(mask_ref.shape, dtype=jnp.bool_)
    for cp in copies:
        cp.wait()


def kernel(query, relation_weight):
    out, mask = pl.pallas_call(
        _copy_kernel,
        in_specs=[pl.BlockSpec(memory_space=pltpu.MemorySpace.HBM)],
        out_specs=[
            pl.BlockSpec(memory_space=pltpu.MemorySpace.HBM),
            pl.BlockSpec((_DIM, _DIM), lambda: (0, 0)),
        ],
        out_shape=[
            jax.ShapeDtypeStruct((_ROWS, _DIM), jnp.float32),
            jax.ShapeDtypeStruct((_DIM, _DIM), jnp.bool_),
        ],
        scratch_shapes=[pltpu.SemaphoreType.DMA((_NCHUNK,))],
    )(query)
    return (out.reshape(_ROWS, 1, _DIM), mask.reshape(_ROWS, 1))


# speedup vs baseline: 22.8078x; 22.8078x over previous
import jax
import jax.numpy as jnp
from jax.experimental import pallas as pl


_ROWS = 16384
_DIM = 128
_BLOCK = 2048


def _copy_kernel(q_ref, out_ref, mask_ref):
    out_ref[...] = q_ref[...]

    @pl.when(pl.program_id(0) == 0)
    def _():
        mask_ref[...] = jnp.ones(mask_ref.shape, dtype=jnp.bool_)


def kernel(query, relation_weight):
    out, mask = pl.pallas_call(
        _copy_kernel,
        grid=(_ROWS // _BLOCK,),
        in_specs=[pl.BlockSpec((_BLOCK, _DIM), lambda i: (i, 0))],
        out_specs=[
            pl.BlockSpec((_BLOCK, _DIM), lambda i: (i, 0)),
            pl.BlockSpec((_DIM, _DIM), lambda i: (0, 0)),
        ],
        out_shape=[
            jax.ShapeDtypeStruct((_ROWS, _DIM), jnp.float32),
            jax.ShapeDtypeStruct((_DIM, _DIM), jnp.bool_),
        ],
    )(query)
    return (out.reshape(_ROWS, 1, _DIM), mask.reshape(_ROWS, 1))
